# speedup vs baseline: 1.6905x; 1.6905x over previous
import functools

import jax
import jax.numpy as jnp
from jax.experimental import pallas as pl
from jax.experimental.pallas import tpu as pltpu

N = 10000
E = 160000
G = 8
D = 300
NB = 25
BLK = N // NB


def _rows(i):
    return (i, 0)


def _full(i):
    return (0, 0)


def _project_body(x_ref, w0, b0, w1, b1, w2, b2, o_ref):
    h = jnp.maximum(x_ref[...] @ w0[...] + b0[...], 0.0)
    h = jnp.maximum(h @ w1[...] + b1[...], 0.0)
    o_ref[...] = h @ w2[...] + b2[...]


def _project(x, p):
    (w0, b0), (w1, b1), (w2, b2) = p
    return pl.pallas_call(
        _project_body,
        grid=(NB,),
        in_specs=[
            pl.BlockSpec((BLK, 3), _rows),
            pl.BlockSpec((3, 64), _full), pl.BlockSpec((64,), lambda i: (0,)),
            pl.BlockSpec((64, 128), _full), pl.BlockSpec((128,), lambda i: (0,)),
            pl.BlockSpec((128, D), _full), pl.BlockSpec((D,), lambda i: (0,)),
        ],
        out_specs=pl.BlockSpec((BLK, D), _rows),
        out_shape=jax.ShapeDtypeStruct((N, D), jnp.float32),
    )(x, w0, b0, w1, b1, w2, b2)


def _pre_body(h_ref, pos_ref, wh, bh, wfp, wfx, s_ref, t_ref):
    h = h_ref[...]
    pos = pos_ref[...]
    delta = h @ wh[...] + bh[...]
    s_ref[...] = pos @ wfp[...] + h @ wfx[...]
    t_ref[...] = (delta - pos) @ wfp[...]


def _conv_pre(h, pos, p):
    (wh, bh), = p['h']
    (wf, _bf), = p['f']
    wfp = wf[:3]
    wfx = wf[3:]
    return pl.pallas_call(
        _pre_body,
        grid=(NB,),
        in_specs=[
            pl.BlockSpec((BLK, D), _rows),
            pl.BlockSpec((BLK, 3), _rows),
            pl.BlockSpec((D, 3), _full), pl.BlockSpec((3,), lambda i: (0,)),
            pl.BlockSpec((3, D), _full),
            pl.BlockSpec((D, D), _full),
        ],
        out_specs=[pl.BlockSpec((BLK, D), _rows), pl.BlockSpec((BLK, D), _rows)],
        out_shape=[jax.ShapeDtypeStruct((N, D), jnp.float32),
                   jax.ShapeDtypeStruct((N, D), jnp.float32)],
    )(h, pos, wh, bh, wfp, wfx)


def _post_body(m_ref, t_ref, h_ref, bf, wg, bg, wl, bl, o_ref):
    m = m_ref[...]
    aggr = jnp.where(jnp.isfinite(m), m + t_ref[...] + bf[...], 0.0)
    out = aggr @ wg[...] + bg[...]
    hr = jnp.maximum(h_ref[...] + out, 0.0)
    o_ref[...] = hr @ wl[...] + bl[...]


def _conv_post(m, t, h, p, plin):
    (_wf, bf), = p['f']
    (wg, bg), = p['g']
    (wl, bl), = plin
    return pl.pallas_call(
        _post_body,
        grid=(NB,),
        in_specs=[
            pl.BlockSpec((BLK, D), _rows),
            pl.BlockSpec((BLK, D), _rows),
            pl.BlockSpec((BLK, D), _rows),
            pl.BlockSpec((D,), lambda i: (0,)),
            pl.BlockSpec((D, D), _full), pl.BlockSpec((D,), lambda i: (0,)),
            pl.BlockSpec((D, D), _full), pl.BlockSpec((D,), lambda i: (0,)),
        ],
        out_specs=pl.BlockSpec((BLK, D), _rows),
        out_shape=jax.ShapeDtypeStruct((N, D), jnp.float32),
    )(m, t, h, bf, wg, bg, wl, bl)


def _head_body(h_ref, onehot_ref, wd, bd, o_ref, acc_s, acc_c):
    i = pl.program_id(0)

    @pl.when(i == 0)
    def _init():
        acc_s[...] = jnp.zeros_like(acc_s)
        acc_c[...] = jnp.zeros_like(acc_c)

    oh = onehot_ref[...]
    acc_s[...] += oh.T @ h_ref[...]
    acc_c[...] += jnp.sum(oh, axis=0)[:, None]

    @pl.when(i == NB - 1)
    def _fin():
        pooled = acc_s[...] / jnp.maximum(acc_c[...], 1.0)
        logits = pooled @ wd[...] + bd[...]
        mx = jnp.max(logits, axis=1, keepdims=True)
        sh = logits - mx
        lse = jnp.log(jnp.sum(jnp.exp(sh), axis=1, keepdims=True))
        o_ref[...] = sh - lse


def _head(h, batch, p):
    (wd, bd), = p
    onehot = (batch[:, None] == jnp.arange(G)[None, :]).astype(jnp.float32)
    return pl.pallas_call(
        _head_body,
        grid=(NB,),
        in_specs=[
            pl.BlockSpec((BLK, D), _rows),
            pl.BlockSpec((BLK, G), _rows),
            pl.BlockSpec((D, 40), _full), pl.BlockSpec((40,), lambda i: (0,)),
        ],
        out_specs=pl.BlockSpec((G, 40), _full),
        out_shape=jax.ShapeDtypeStruct((G, 40), jnp.float32),
        scratch_shapes=[pltpu.VMEM((G, D), jnp.float32),
                        pltpu.VMEM((G, 1), jnp.float32)],
    )(h, onehot, wd, bd)


def _segment_max(s, src, dst):
    g = s[src]
    return jax.ops.segment_max(g, dst, num_segments=N)


def kernel(x, edge_index, batch, params):
    src = edge_index[0]
    dst = edge_index[1]
    pos = x
    h = _project(x, params['project'])
    for c, l in (('conv1', 'lin1'), ('conv2', 'lin2'), ('conv3', 'lin3')):
        p = params[c]
        s, t = _conv_pre(h, pos, p)
        m = _segment_max(s, src, dst)
        h = _conv_post(m, t, h, p, params[l])
    return _head(h, batch, params['decision'])
